# baseline (speedup 1.0000x reference)
import functools

import jax
import jax.numpy as jnp
from jax import lax
from jax.experimental import pallas as pl
from jax.experimental.pallas import tpu as pltpu
from jax.experimental.pallas import tpu_sc as plsc

B, S, D = 4096, 50, 128
V, C = 1000000, 1000
CPAD = 1024

NC, NS = 2, 16
NW = NC * NS
BAGS_W = B // NW
TOK_W = BAGS_W * S
CHUNK = 128
NCHUNK = TOK_W // CHUNK


def _pool_sc(emb_table, enc_w, bag_idx, zeros):
    mesh = plsc.VectorSubcoreMesh(core_axis_name="c", subcore_axis_name="s")

    @functools.partial(
        pl.kernel,
        out_type=jax.ShapeDtypeStruct((NW, BAGS_W, D), jnp.float32),
        mesh=mesh,
        scratch_types=[
            pltpu.VMEM((NCHUNK, CHUNK), jnp.int32),
            pltpu.VMEM((NCHUNK, CHUNK), jnp.int32),
            pltpu.VMEM((2, CHUNK, D), jnp.float32),
            pltpu.VMEM_SHARED((NS * BAGS_W, D), jnp.float32),
            pltpu.SemaphoreType.DMA,
            pltpu.SemaphoreType.DMA,
        ],
    )
    def k(table_hbm, enc_hbm, bag_hbm, zero_hbm, out_hbm,
          idx_v, bag_v, rows_v, pooled_sh, sem0, sem1):
        sid = lax.axis_index("s")
        wid = sid * NC + lax.axis_index("c")
        pltpu.sync_copy(enc_hbm.at[wid], idx_v)
        pltpu.sync_copy(bag_hbm.at[sid], bag_v)
        pltpu.sync_copy(zero_hbm, pooled_sh.at[pl.ds(sid * BAGS_W, BAGS_W)])
        sems = (sem0, sem1)

        def start(c, b):
            pltpu.async_copy(table_hbm.at[idx_v.at[c]], rows_v.at[b], sems[b])

        def finish(c, b):
            pltpu.make_async_copy(table_hbm.at[idx_v.at[c]], rows_v.at[b],
                                  sems[b]).wait()
            pltpu.sync_copy(rows_v.at[b], pooled_sh.at[bag_v.at[c]], add=True)

        start(0, 0)
        start(1, 1)

        @pl.loop(0, NCHUNK - 2, step=2)
        def _(c2):
            for b in range(2):
                finish(c2 + b, b)
                start(c2 + b + 2, b)

        for b in range(2):
            finish(NCHUNK - 2 + b, b)

        pltpu.sync_copy(pooled_sh.at[pl.ds(sid * BAGS_W, BAGS_W)],
                        out_hbm.at[wid])

    return k(emb_table, enc_w, bag_idx, zeros)


def _head_tc(pooled, enc, fc_w, fc_b2):
    BM = 512

    def body(x_ref, enc_ref, w_ref, b_ref, out_ref):
        cnt = jnp.sum((enc_ref[...] != 0).astype(jnp.float32), axis=1,
                      keepdims=True)
        x = x_ref[...] / jnp.maximum(cnt, 1.0)
        out_ref[...] = lax.dot_general(
            x, w_ref[...], (((1,), (1,)), ((), ())),
            preferred_element_type=jnp.float32) + b_ref[...]

    return pl.pallas_call(
        body,
        grid=(B // BM,),
        in_specs=[
            pl.BlockSpec((BM, D), lambda i: (i, 0)),
            pl.BlockSpec((BM, S), lambda i: (i, 0)),
            pl.BlockSpec((C, D), lambda i: (0, 0)),
            pl.BlockSpec((1, C), lambda i: (0, 0)),
        ],
        out_specs=pl.BlockSpec((BM, C), lambda i: (i, 0)),
        out_shape=jax.ShapeDtypeStruct((B, C), jnp.float32),
    )(pooled, enc, fc_w, fc_b2)


def kernel(encoded_text, additional_inputs, emb_table, fc_w, fc_b):
    del additional_inputs
    enc = encoded_text.astype(jnp.int32)
    enc_w = enc.reshape(NW, NCHUNK, CHUNK)
    bag_local = (jnp.arange(TOK_W, dtype=jnp.int32) // S).reshape(NCHUNK, CHUNK)
    bag_idx = (bag_local[None] +
               BAGS_W * jnp.arange(NS, dtype=jnp.int32)[:, None, None])
    zeros = jnp.zeros((BAGS_W, D), jnp.float32)
    pooled = _pool_sc(emb_table, enc_w, bag_idx, zeros).reshape(B, D)
    return _head_tc(pooled, enc, fc_w, fc_b.reshape(1, C))


# speedup vs baseline: 1.3075x; 1.3075x over previous
import functools

import jax
import jax.numpy as jnp
from jax import lax
from jax.experimental import pallas as pl
from jax.experimental.pallas import tpu as pltpu
from jax.experimental.pallas import tpu_sc as plsc

B, S, D = 4096, 50, 128
V, C = 1000000, 1000

NC, NS = 2, 16
NW = NC * NS
BAGS_W = B // NW
CHUNK = 128


def _pool_sc(emb_table, enc3, bag_idx, zeros):
    mesh = plsc.VectorSubcoreMesh(core_axis_name="c", subcore_axis_name="s")

    @functools.partial(
        pl.kernel,
        out_type=jax.ShapeDtypeStruct((NW, BAGS_W, D), jnp.float32),
        mesh=mesh,
        scratch_types=[
            pltpu.VMEM((S, CHUNK), jnp.int32),
            pltpu.VMEM((1, CHUNK), jnp.int32),
            pltpu.VMEM((2, CHUNK, D), jnp.float32),
            pltpu.VMEM_SHARED((NS * BAGS_W, D), jnp.float32),
            pltpu.SemaphoreType.DMA,
            pltpu.SemaphoreType.DMA,
        ],
    )
    def k(table_hbm, enc_hbm, bag_hbm, zero_hbm, out_hbm,
          idx_v, bag_v, rows_v, pooled_sh, sem0, sem1):
        sid = lax.axis_index("s")
        wid = sid * NC + lax.axis_index("c")
        pltpu.sync_copy(enc_hbm.at[:, wid], idx_v)
        pltpu.sync_copy(bag_hbm.at[pl.ds(sid, 1)], bag_v)
        pltpu.sync_copy(zero_hbm, pooled_sh.at[pl.ds(sid * BAGS_W, BAGS_W)])
        sems = (sem0, sem1)

        def start(c, b):
            pltpu.async_copy(table_hbm.at[idx_v.at[c]], rows_v.at[b], sems[b])

        def finish(c, b):
            pltpu.make_async_copy(table_hbm.at[idx_v.at[c]], rows_v.at[b],
                                  sems[b]).wait()
            pltpu.sync_copy(rows_v.at[b], pooled_sh.at[bag_v.at[0]], add=True)

        start(0, 0)
        start(1, 1)

        @pl.loop(0, S - 2, step=2)
        def _(c2):
            for b in range(2):
                finish(c2 + b, b)
                start(c2 + b + 2, b)

        for b in range(2):
            finish(S - 2 + b, b)

        pltpu.sync_copy(pooled_sh.at[pl.ds(sid * BAGS_W, BAGS_W)],
                        out_hbm.at[wid])

    return k(emb_table, enc3, bag_idx, zeros)


def _head_tc(pooled, enc_t, fc_w, fc_b_col):
    BM = 512

    def body(x_ref, enc_ref, w_ref, b_ref, out_ref):
        cnt = jnp.sum((enc_ref[...] != 0).astype(jnp.float32), axis=0,
                      keepdims=True)
        scale = 1.0 / jnp.maximum(cnt, 1.0)
        acc = lax.dot_general(
            w_ref[...], x_ref[...], (((1,), (1,)), ((), ())),
            preferred_element_type=jnp.float32)
        out_ref[...] = acc * scale + b_ref[...]

    return pl.pallas_call(
        body,
        grid=(B // BM,),
        in_specs=[
            pl.BlockSpec((BM, D), lambda i: (i, 0)),
            pl.BlockSpec((S, BM), lambda i: (0, i)),
            pl.BlockSpec((C, D), lambda i: (0, 0)),
            pl.BlockSpec((C, 1), lambda i: (0, 0)),
        ],
        out_specs=pl.BlockSpec((C, BM), lambda i: (0, i)),
        out_shape=jax.ShapeDtypeStruct((C, B), jnp.float32),
    )(pooled, enc_t, fc_w, fc_b_col)


def kernel(encoded_text, additional_inputs, emb_table, fc_w, fc_b):
    del additional_inputs
    enc_t = encoded_text.astype(jnp.int32).T
    enc3 = enc_t.reshape(S, NW, CHUNK)
    bag_idx = (jnp.arange(CHUNK, dtype=jnp.int32)[None, :] +
               BAGS_W * jnp.arange(NS, dtype=jnp.int32)[:, None])
    zeros = jnp.zeros((BAGS_W, D), jnp.float32)
    pooled = _pool_sc(emb_table, enc3, bag_idx, zeros).reshape(B, D)
    out_t = _head_tc(pooled, enc_t, fc_w, fc_b.reshape(C, 1))
    return out_t.T
